# Initial kernel scaffold; baseline (speedup 1.0000x reference)
import jax
import jax.numpy as jnp
from jax.experimental import pallas as pl


def kernel(init_node_features, edge_index, W1, b1, W2, b2):
    raise NotImplementedError("write your pallas kernel here")


# speedup vs baseline: 24.8178x; 24.8178x over previous
import functools

import jax
import jax.numpy as jnp
from jax import lax
from jax.experimental import pallas as pl
from jax.experimental.pallas import tpu as pltpu
from jax.experimental.pallas import tpu_sc as plsc

N = 10000
NPAD = 10240
E = 320000
EPAD = 327680
EBLK = 128
CHUNKS = EPAD // (32 * EBLK)
STRIPE = NPAD // 16
D = 128
DMID = 256

_MESH = dict(core_axis_name="c", subcore_axis_name="s", num_cores=2,
             num_subcores=16)


def _zero_rows_v(rows_v):
  z16 = jnp.zeros((16,), jnp.float32)

  @pl.loop(0, EBLK)
  def _(i):
    for j in range(8):
      rows_v[i, pl.ds(16 * j, 16)] = z16


def _agg_kernel(table, rows_h, cols_h, out0, out1, row_t, col_t, rows_v, sem,
                acc):
  c = lax.axis_index("c")
  s = lax.axis_index("s")
  w = c * 16 + s

  _zero_rows_v(rows_v)
  for k in range(STRIPE // EBLK):
    pltpu.sync_copy(rows_v, acc.at[pl.ds(s * STRIPE + k * EBLK, EBLK)])
  plsc.subcore_barrier()

  pltpu.sync_copy(rows_h.at[pl.ds(w * CHUNKS, CHUNKS)], row_t)
  pltpu.sync_copy(cols_h.at[pl.ds(w * CHUNKS, CHUNKS)], col_t)

  @pl.loop(0, CHUNKS)
  def _(b):
    pltpu.async_copy(table.at[row_t.at[b]], rows_v, sem).wait()
    pltpu.sync_copy(rows_v, acc.at[col_t.at[b]], add=True)

  plsc.subcore_barrier()

  @pl.when(c == 0)
  def _():
    pltpu.sync_copy(acc.at[pl.ds(s * STRIPE, STRIPE)],
                    out0.at[pl.ds(s * STRIPE, STRIPE)])

  @pl.when(c == 1)
  def _():
    pltpu.sync_copy(acc.at[pl.ds(s * STRIPE, STRIPE)],
                    out1.at[pl.ds(s * STRIPE, STRIPE)])


@functools.cache
def _agg():
  return pl.kernel(
      _agg_kernel,
      out_type=(jax.ShapeDtypeStruct((NPAD, D), jnp.float32),
                jax.ShapeDtypeStruct((NPAD, D), jnp.float32)),
      mesh=plsc.VectorSubcoreMesh(**_MESH),
      scratch_types=[
          pltpu.VMEM((CHUNKS, EBLK), jnp.int32),
          pltpu.VMEM((CHUNKS, EBLK), jnp.int32),
          pltpu.VMEM((EBLK, D), jnp.float32),
          pltpu.SemaphoreType.DMA,
          pltpu.VMEM_SHARED((NPAD, D), jnp.float32),
      ],
  )


def _deg_kernel(cols_h, out0, out1, col_t, ones_v, zeros_v, acc):
  c = lax.axis_index("c")
  s = lax.axis_index("s")
  w = c * 16 + s

  o16 = jnp.ones((16,), jnp.float32)
  z16 = jnp.zeros((16,), jnp.float32)
  for j in range(EBLK // 16):
    ones_v[pl.ds(16 * j, 16)] = o16

  @pl.loop(0, STRIPE // 16)
  def _(i):
    zeros_v[pl.ds(16 * i, 16)] = z16

  pltpu.sync_copy(zeros_v, acc.at[pl.ds(s * STRIPE, STRIPE)])
  plsc.subcore_barrier()

  pltpu.sync_copy(cols_h.at[pl.ds(w * CHUNKS, CHUNKS)], col_t)

  @pl.loop(0, CHUNKS)
  def _(b):
    pltpu.sync_copy(ones_v, acc.at[col_t.at[b]], add=True)

  plsc.subcore_barrier()

  @pl.when(c == 0)
  def _():
    pltpu.sync_copy(acc.at[pl.ds(s * STRIPE, STRIPE)],
                    out0.at[pl.ds(s * STRIPE, STRIPE)])

  @pl.when(c == 1)
  def _():
    pltpu.sync_copy(acc.at[pl.ds(s * STRIPE, STRIPE)],
                    out1.at[pl.ds(s * STRIPE, STRIPE)])


@functools.cache
def _deg():
  return pl.kernel(
      _deg_kernel,
      out_type=(jax.ShapeDtypeStruct((NPAD,), jnp.float32),
                jax.ShapeDtypeStruct((NPAD,), jnp.float32)),
      mesh=plsc.VectorSubcoreMesh(**_MESH),
      scratch_types=[
          pltpu.VMEM((CHUNKS, EBLK), jnp.int32),
          pltpu.VMEM((EBLK,), jnp.float32),
          pltpu.VMEM((STRIPE,), jnp.float32),
          pltpu.VMEM_SHARED((NPAD,), jnp.float32),
      ],
  )


_RB = 1024


def _scale_body(deg0, deg1, x, xs, dinv):
  d = deg0[...] + deg1[...] + 1.0
  di = lax.rsqrt(d)
  xs[...] = x[...] * di
  dinv[...] = di


def _scale(deg0, deg1, x):
  return pl.pallas_call(
      _scale_body,
      grid=(NPAD // _RB,),
      in_specs=[
          pl.BlockSpec((_RB, 1), lambda i: (i, 0)),
          pl.BlockSpec((_RB, 1), lambda i: (i, 0)),
          pl.BlockSpec((_RB, D), lambda i: (i, 0)),
      ],
      out_specs=[
          pl.BlockSpec((_RB, D), lambda i: (i, 0)),
          pl.BlockSpec((_RB, 1), lambda i: (i, 0)),
      ],
      out_shape=[
          jax.ShapeDtypeStruct((NPAD, D), jnp.float32),
          jax.ShapeDtypeStruct((NPAD, 1), jnp.float32),
      ],
  )(deg0, deg1, x)


def _mid_body(a0, a1, xs, dinv, w1, b1, w2, hs2):
  i = pl.program_id(0)
  di = dinv[...]
  p1 = (a0[...] + a1[...] + xs[...]) * di
  h1 = jnp.maximum(jnp.dot(p1, w1[...]) + b1[...], 0.0)
  rid = i * _RB + lax.broadcasted_iota(jnp.int32, (_RB, 1), 0)
  h1 = jnp.where(rid < N, h1, 0.0)
  hs2[...] = jnp.dot(h1, w2[...]) * di


def _mid(a0, a1, xs, dinv, w1, b1, w2):
  return pl.pallas_call(
      _mid_body,
      grid=(NPAD // _RB,),
      in_specs=[
          pl.BlockSpec((_RB, D), lambda i: (i, 0)),
          pl.BlockSpec((_RB, D), lambda i: (i, 0)),
          pl.BlockSpec((_RB, D), lambda i: (i, 0)),
          pl.BlockSpec((_RB, 1), lambda i: (i, 0)),
          pl.BlockSpec((D, DMID), lambda i: (0, 0)),
          pl.BlockSpec((1, DMID), lambda i: (0, 0)),
          pl.BlockSpec((DMID, D), lambda i: (0, 0)),
      ],
      out_specs=pl.BlockSpec((_RB, D), lambda i: (i, 0)),
      out_shape=jax.ShapeDtypeStruct((NPAD, D), jnp.float32),
  )(a0, a1, xs, dinv, w1, b1, w2)


_RBF = 400


def _final_body(c0, c1, hs2, dinv, b2, out):
  h2 = (c0[...] + c1[...] + hs2[...]) * dinv[...] + b2[...]
  m = jnp.max(h2, axis=1, keepdims=True)
  ex = jnp.exp(h2 - m)
  lse = jnp.log(jnp.sum(ex, axis=1, keepdims=True)) + m
  out[...] = h2 - lse


def _final(c0, c1, hs2, dinv, b2):
  return pl.pallas_call(
      _final_body,
      grid=(N // _RBF,),
      in_specs=[
          pl.BlockSpec((_RBF, D), lambda i: (i, 0)),
          pl.BlockSpec((_RBF, D), lambda i: (i, 0)),
          pl.BlockSpec((_RBF, D), lambda i: (i, 0)),
          pl.BlockSpec((_RBF, 1), lambda i: (i, 0)),
          pl.BlockSpec((1, D), lambda i: (0, 0)),
      ],
      out_specs=pl.BlockSpec((_RBF, D), lambda i: (i, 0)),
      out_shape=jax.ShapeDtypeStruct((N, D), jnp.float32),
  )(c0, c1, hs2, dinv, b2)


def kernel(init_node_features, edge_index, W1, b1, W2, b2):
  row = edge_index[0].astype(jnp.int32)
  col = edge_index[1].astype(jnp.int32)
  pad = N + (jnp.arange(EPAD - E, dtype=jnp.int32) % (NPAD - N))
  rows_h = jnp.concatenate([row, pad]).reshape(32 * CHUNKS, EBLK)
  cols_h = jnp.concatenate([col, pad]).reshape(32 * CHUNKS, EBLK)
  x = jnp.pad(init_node_features, ((0, NPAD - N), (0, 0)))

  deg0, deg1 = _deg()(cols_h)
  xs, dinv = _scale(deg0.reshape(NPAD, 1), deg1.reshape(NPAD, 1), x)
  a0, a1 = _agg()(xs, rows_h, cols_h)
  hs2 = _mid(a0, a1, xs, dinv, W1, b1.reshape(1, DMID), W2)
  c0, c1 = _agg()(hs2, rows_h, cols_h)
  return _final(c0, c1, hs2, dinv, b2.reshape(1, D))


# speedup vs baseline: 35.4629x; 1.4289x over previous
import functools

import jax
import jax.numpy as jnp
from jax import lax
from jax.experimental import pallas as pl
from jax.experimental.pallas import tpu as pltpu
from jax.experimental.pallas import tpu_sc as plsc

N = 10000
NPAD = 10240
E = 320000
EPAD = 327680
EBLK = 128
CHUNKS = EPAD // (32 * EBLK)
PBLK = CHUNKS // 2
STRIPE = NPAD // 16
D = 128
DMID = 256

_MESH = dict(core_axis_name="c", subcore_axis_name="s", num_cores=2,
             num_subcores=16)


def _zero_rows_v(rows_v):
  z16 = jnp.zeros((16,), jnp.float32)

  @pl.loop(0, EBLK)
  def _(i):
    for j in range(8):
      rows_v[i, pl.ds(16 * j, 16)] = z16


def _agg_kernel(table, rows_h, cols_h, out0, out1, row_t, col_t, rows_v0,
                rows_v1, sem0, sem1, acc):
  c = lax.axis_index("c")
  s = lax.axis_index("s")
  w = c * 16 + s
  bufs = (rows_v0, rows_v1)
  sems = (sem0, sem1)

  _zero_rows_v(rows_v0)
  for k in range(STRIPE // EBLK):
    pltpu.sync_copy(rows_v0, acc.at[pl.ds(s * STRIPE + k * EBLK, EBLK)])
  plsc.subcore_barrier()

  for phase in range(CHUNKS // PBLK):
    base = w * CHUNKS + phase * PBLK
    pltpu.sync_copy(rows_h.at[pl.ds(base, PBLK)], row_t)
    pltpu.sync_copy(cols_h.at[pl.ds(base, PBLK)], col_t)
    pltpu.async_copy(table.at[row_t.at[0]], rows_v0, sem0)

    @pl.loop(0, PBLK // 2)
    def _(i):
      for k in range(2):
        b = 2 * i + k

        @pl.when(b + 1 < PBLK)
        def _():
          pltpu.async_copy(table.at[row_t.at[b + 1]], bufs[1 - k],
                           sems[1 - k])

        pltpu.make_async_copy(table.at[row_t.at[b]], bufs[k], sems[k]).wait()
        pltpu.sync_copy(bufs[k], acc.at[col_t.at[b]], add=True)

  plsc.subcore_barrier()

  @pl.when(c == 0)
  def _():
    pltpu.sync_copy(acc.at[pl.ds(s * STRIPE, STRIPE)],
                    out0.at[pl.ds(s * STRIPE, STRIPE)])

  @pl.when(c == 1)
  def _():
    pltpu.sync_copy(acc.at[pl.ds(s * STRIPE, STRIPE)],
                    out1.at[pl.ds(s * STRIPE, STRIPE)])


@functools.cache
def _agg():
  return pl.kernel(
      _agg_kernel,
      out_type=(jax.ShapeDtypeStruct((NPAD, D), jnp.float32),
                jax.ShapeDtypeStruct((NPAD, D), jnp.float32)),
      mesh=plsc.VectorSubcoreMesh(**_MESH),
      scratch_types=[
          pltpu.VMEM((PBLK, EBLK), jnp.int32),
          pltpu.VMEM((PBLK, EBLK), jnp.int32),
          pltpu.VMEM((EBLK, D), jnp.float32),
          pltpu.VMEM((EBLK, D), jnp.float32),
          pltpu.SemaphoreType.DMA,
          pltpu.SemaphoreType.DMA,
          pltpu.VMEM_SHARED((NPAD, D), jnp.float32),
      ],
  )


def _deg_kernel(cols_h, out0, out1, col_t, ones_v, zeros_v, acc):
  c = lax.axis_index("c")
  s = lax.axis_index("s")
  w = c * 16 + s

  o16 = jnp.ones((16,), jnp.float32)
  z16 = jnp.zeros((16,), jnp.float32)
  for j in range(EBLK // 16):
    ones_v[pl.ds(16 * j, 16)] = o16

  @pl.loop(0, STRIPE // 16)
  def _(i):
    zeros_v[pl.ds(16 * i, 16)] = z16

  pltpu.sync_copy(zeros_v, acc.at[pl.ds(s * STRIPE, STRIPE)])
  plsc.subcore_barrier()

  pltpu.sync_copy(cols_h.at[pl.ds(w * CHUNKS, CHUNKS)], col_t)

  @pl.loop(0, CHUNKS)
  def _(b):
    pltpu.sync_copy(ones_v, acc.at[col_t.at[b]], add=True)

  plsc.subcore_barrier()

  @pl.when(c == 0)
  def _():
    pltpu.sync_copy(acc.at[pl.ds(s * STRIPE, STRIPE)],
                    out0.at[pl.ds(s * STRIPE, STRIPE)])

  @pl.when(c == 1)
  def _():
    pltpu.sync_copy(acc.at[pl.ds(s * STRIPE, STRIPE)],
                    out1.at[pl.ds(s * STRIPE, STRIPE)])


@functools.cache
def _deg():
  return pl.kernel(
      _deg_kernel,
      out_type=(jax.ShapeDtypeStruct((NPAD,), jnp.float32),
                jax.ShapeDtypeStruct((NPAD,), jnp.float32)),
      mesh=plsc.VectorSubcoreMesh(**_MESH),
      scratch_types=[
          pltpu.VMEM((CHUNKS, EBLK), jnp.int32),
          pltpu.VMEM((EBLK,), jnp.float32),
          pltpu.VMEM((STRIPE,), jnp.float32),
          pltpu.VMEM_SHARED((NPAD,), jnp.float32),
      ],
  )


_RB = 1024


def _scale_body(deg0, deg1, x, xs, dinv):
  d = deg0[...] + deg1[...] + 1.0
  di = lax.rsqrt(d)
  xs[...] = x[...] * di
  dinv[...] = di


def _scale(deg0, deg1, x):
  return pl.pallas_call(
      _scale_body,
      grid=(NPAD // _RB,),
      in_specs=[
          pl.BlockSpec((_RB, 1), lambda i: (i, 0)),
          pl.BlockSpec((_RB, 1), lambda i: (i, 0)),
          pl.BlockSpec((_RB, D), lambda i: (i, 0)),
      ],
      out_specs=[
          pl.BlockSpec((_RB, D), lambda i: (i, 0)),
          pl.BlockSpec((_RB, 1), lambda i: (i, 0)),
      ],
      out_shape=[
          jax.ShapeDtypeStruct((NPAD, D), jnp.float32),
          jax.ShapeDtypeStruct((NPAD, 1), jnp.float32),
      ],
  )(deg0, deg1, x)


def _mid_body(a0, a1, xs, dinv, w1, b1, w2, hs2):
  i = pl.program_id(0)
  di = dinv[...]
  p1 = (a0[...] + a1[...] + xs[...]) * di
  h1 = jnp.maximum(jnp.dot(p1, w1[...]) + b1[...], 0.0)
  rid = i * _RB + lax.broadcasted_iota(jnp.int32, (_RB, 1), 0)
  h1 = jnp.where(rid < N, h1, 0.0)
  hs2[...] = jnp.dot(h1, w2[...]) * di


def _mid(a0, a1, xs, dinv, w1, b1, w2):
  return pl.pallas_call(
      _mid_body,
      grid=(NPAD // _RB,),
      in_specs=[
          pl.BlockSpec((_RB, D), lambda i: (i, 0)),
          pl.BlockSpec((_RB, D), lambda i: (i, 0)),
          pl.BlockSpec((_RB, D), lambda i: (i, 0)),
          pl.BlockSpec((_RB, 1), lambda i: (i, 0)),
          pl.BlockSpec((D, DMID), lambda i: (0, 0)),
          pl.BlockSpec((1, DMID), lambda i: (0, 0)),
          pl.BlockSpec((DMID, D), lambda i: (0, 0)),
      ],
      out_specs=pl.BlockSpec((_RB, D), lambda i: (i, 0)),
      out_shape=jax.ShapeDtypeStruct((NPAD, D), jnp.float32),
  )(a0, a1, xs, dinv, w1, b1, w2)


_RBF = 400


def _final_body(c0, c1, hs2, dinv, b2, out):
  h2 = (c0[...] + c1[...] + hs2[...]) * dinv[...] + b2[...]
  m = jnp.max(h2, axis=1, keepdims=True)
  ex = jnp.exp(h2 - m)
  lse = jnp.log(jnp.sum(ex, axis=1, keepdims=True)) + m
  out[...] = h2 - lse


def _final(c0, c1, hs2, dinv, b2):
  return pl.pallas_call(
      _final_body,
      grid=(N // _RBF,),
      in_specs=[
          pl.BlockSpec((_RBF, D), lambda i: (i, 0)),
          pl.BlockSpec((_RBF, D), lambda i: (i, 0)),
          pl.BlockSpec((_RBF, D), lambda i: (i, 0)),
          pl.BlockSpec((_RBF, 1), lambda i: (i, 0)),
          pl.BlockSpec((1, D), lambda i: (0, 0)),
      ],
      out_specs=pl.BlockSpec((_RBF, D), lambda i: (i, 0)),
      out_shape=jax.ShapeDtypeStruct((N, D), jnp.float32),
  )(c0, c1, hs2, dinv, b2)


def kernel(init_node_features, edge_index, W1, b1, W2, b2):
  row = edge_index[0].astype(jnp.int32)
  col = edge_index[1].astype(jnp.int32)
  pad = N + (jnp.arange(EPAD - E, dtype=jnp.int32) % (NPAD - N))
  rows_h = jnp.concatenate([row, pad]).reshape(32 * CHUNKS, EBLK)
  cols_h = jnp.concatenate([col, pad]).reshape(32 * CHUNKS, EBLK)
  x = jnp.pad(init_node_features, ((0, NPAD - N), (0, 0)))

  deg0, deg1 = _deg()(cols_h)
  xs, dinv = _scale(deg0.reshape(NPAD, 1), deg1.reshape(NPAD, 1), x)
  a0, a1 = _agg()(xs, rows_h, cols_h)
  hs2 = _mid(a0, a1, xs, dinv, W1, b1.reshape(1, DMID), W2)
  c0, c1 = _agg()(hs2, rows_h, cols_h)
  return _final(c0, c1, hs2, dinv, b2.reshape(1, D))
